# Initial kernel scaffold; baseline (speedup 1.0000x reference)
import jax
import jax.numpy as jnp
from jax.experimental import pallas as pl


def kernel(relative_position_bias_table, window_size):
    raise NotImplementedError("write your pallas kernel here")


# speedup vs baseline: 5.2095x; 5.2095x over previous
import functools

import jax
import jax.numpy as jnp
from jax import lax
from jax.experimental import pallas as pl
from jax.experimental.pallas import tpu as pltpu
from jax.experimental.pallas import tpu_sc as plsc

NUM_HEADS = 16
SEQ = 1025
TBL_WORDS = 3972 * NUM_HEADS
NW = 32
TOTAL_ROWS = NUM_HEADS * SEQ
ROW_PAD = 1040

T_ROW0 = 3969 * 16
T_COL0 = 3970 * 16
T_CORNER = 3971 * 16


@functools.partial(
    pl.kernel,
    mesh=plsc.VectorSubcoreMesh(core_axis_name="c", subcore_axis_name="s"),
    out_type=jax.ShapeDtypeStruct((NUM_HEADS, SEQ, SEQ), jnp.float32),
    scratch_types=[
        pltpu.VMEM((TBL_WORDS,), jnp.float32),
        pltpu.VMEM((ROW_PAD,), jnp.float32),
    ],
    compiler_params=pltpu.CompilerParams(
        needs_layout_passes=False, use_tc_tiling_on_sc=False),
)
def _bias_kernel(tbl_hbm, out_hbm, tbl_v, row_v):
    wid = lax.axis_index("s") * 2 + lax.axis_index("c")
    pltpu.sync_copy(tbl_hbm, tbl_v)

    start = (wid * TOTAL_ROWS) // NW
    end = ((wid + 1) * TOTAL_ROWS) // NW
    lane = lax.iota(jnp.int32, 16)
    iota16 = lane * 16
    lane0 = lane == 0
    edge = jnp.where(lane0, 496, 0).astype(jnp.int32)

    def do_row(r, carry):
        h = r // SEQ
        i = r - h * SEQ

        @pl.when(i == 0)
        def _():
            vspec = plsc.load_gather(
                tbl_v, [jnp.where(lane0, T_CORNER + h, T_ROW0 + h)])
            row_v[pl.ds(0, 16)] = vspec
            vfill = plsc.load_gather(
                tbl_v, [jnp.full((16,), T_ROW0 + h, jnp.int32)])
            for k in range(1, 65):
                row_v[pl.ds(16 * k, 16)] = vfill

        @pl.when(i != 0)
        def _():
            p = i - 1
            r0 = p // 32
            c0 = p - r0 * 32
            s = ((r0 + 31) * 63 + (c0 + 31)) * 16 + h
            base0 = jnp.full((16,), s + 16, jnp.int32) - iota16
            o0 = jnp.where(lane0, T_COL0 + h, base0)
            row_v[pl.ds(0, 16)] = plsc.load_gather(tbl_v, [o0])
            for k in range(1, 64):
                if k % 2 == 1:
                    off = base0 - (1008 * ((k - 1) // 2) + 256)
                else:
                    off = base0 - 1008 * (k // 2) + edge
                row_v[pl.ds(16 * k, 16)] = plsc.load_gather(tbl_v, [off])
            o64 = jnp.full((16,), s - 31744, jnp.int32)
            row_v[pl.ds(1024, 16)] = plsc.load_gather(tbl_v, [o64])

        pltpu.sync_copy(row_v.at[pl.ds(0, SEQ)], out_hbm.at[h, i])
        return carry

    lax.fori_loop(start, end, do_row, 0)


def kernel(relative_position_bias_table, window_size):
    del window_size
    tbl_flat = relative_position_bias_table.reshape(-1)
    out = _bias_kernel(tbl_flat)
    return out[None]


# speedup vs baseline: 5.5152x; 1.0587x over previous
import functools

import jax
import jax.numpy as jnp
from jax import lax
from jax.experimental import pallas as pl
from jax.experimental.pallas import tpu as pltpu
from jax.experimental.pallas import tpu_sc as plsc

NUM_HEADS = 16
SEQ = 1025
TBL_WORDS = 3972 * NUM_HEADS
NW = 32
TOTAL_ROWS = NUM_HEADS * SEQ
NB = 4
ROWS_PER = 520
ROW_PAD = 1040

T_ROW0 = 3969 * 16
T_COL0 = 3970 * 16
T_CORNER = 3971 * 16


@functools.partial(
    pl.kernel,
    mesh=plsc.VectorSubcoreMesh(core_axis_name="c", subcore_axis_name="s"),
    out_type=jax.ShapeDtypeStruct((NUM_HEADS, SEQ, SEQ), jnp.float32),
    scratch_types=(
        [pltpu.VMEM((TBL_WORDS,), jnp.float32)]
        + [pltpu.VMEM((ROW_PAD,), jnp.float32)] * NB
        + [pltpu.SemaphoreType.DMA] * NB
    ),
    compiler_params=pltpu.CompilerParams(
        needs_layout_passes=False, use_tc_tiling_on_sc=False),
)
def _bias_kernel(tbl_hbm, out_hbm, tbl_v, *bufs_sems):
    bufs, sems = bufs_sems[:NB], bufs_sems[NB:]
    wid = lax.axis_index("s") * 2 + lax.axis_index("c")
    pltpu.sync_copy(tbl_hbm, tbl_v)

    start = (wid * (TOTAL_ROWS - ROWS_PER)) // (NW - 1)
    lane = lax.iota(jnp.int32, 16)
    iota16 = lane * 16
    lane0 = lane == 0
    edge = jnp.where(lane0, 496, 0).astype(jnp.int32)

    def build_row(buf, r):
        h = r // SEQ
        i = r - h * SEQ

        @pl.when(i == 0)
        def _():
            vspec = plsc.load_gather(
                tbl_v, [jnp.where(lane0, T_CORNER + h, T_ROW0 + h)])
            buf[pl.ds(0, 16)] = vspec
            vfill = plsc.load_gather(
                tbl_v, [jnp.full((16,), T_ROW0 + h, jnp.int32)])
            for j in range(1, 65):
                buf[pl.ds(16 * j, 16)] = vfill

        @pl.when(i != 0)
        def _():
            p = i - 1
            r0 = p // 32
            c0 = p - r0 * 32
            s = ((r0 + 31) * 63 + (c0 + 31)) * 16 + h
            base0 = jnp.full((16,), s + 16, jnp.int32) - iota16
            o0 = jnp.where(lane0, T_COL0 + h, base0)
            buf[pl.ds(0, 16)] = plsc.load_gather(tbl_v, [o0])
            for j in range(1, 64):
                if j % 2 == 1:
                    off = base0 - (1008 * ((j - 1) // 2) + 256)
                else:
                    off = base0 - 1008 * (j // 2) + edge
                buf[pl.ds(16 * j, 16)] = plsc.load_gather(tbl_v, [off])
            o64 = jnp.full((16,), s - 31744, jnp.int32)
            buf[pl.ds(1024, 16)] = plsc.load_gather(tbl_v, [o64])

    def do_iter(it, carry):
        for b in range(NB):
            buf, sem = bufs[b], sems[b]

            @pl.when(it > 0)
            def _():
                pltpu.make_async_copy(
                    buf.at[pl.ds(0, SEQ)], out_hbm.at[0, 0], sem).wait()

            r = start + NB * it + b
            build_row(buf, r)
            pltpu.async_copy(
                buf.at[pl.ds(0, SEQ)], out_hbm.at[r // SEQ, r % SEQ], sem)
        return carry

    lax.fori_loop(0, ROWS_PER // NB, do_iter, 0)
    for b in range(NB):
        pltpu.make_async_copy(
            bufs[b].at[pl.ds(0, SEQ)], out_hbm.at[0, 0], sems[b]).wait()


def kernel(relative_position_bias_table, window_size):
    del window_size
    tbl_flat = relative_position_bias_table.reshape(-1)
    out = _bias_kernel(tbl_flat)
    return out[None]


# speedup vs baseline: 6.6770x; 1.2107x over previous
import functools

import jax
import jax.numpy as jnp
from jax import lax
from jax.experimental import pallas as pl
from jax.experimental.pallas import tpu as pltpu
from jax.experimental.pallas import tpu_sc as plsc

NUM_HEADS = 16
SEQ = 1025
TBL_WORDS = 3972 * NUM_HEADS
NW = 32
TOTAL_ROWS = NUM_HEADS * SEQ
NB = 4
ROWS_PER = 520
ROW_PAD = 1040

T_ROW0 = 3969
T_COL0 = 3970
T_CORNER = 3971


@functools.partial(
    pl.kernel,
    mesh=plsc.VectorSubcoreMesh(core_axis_name="c", subcore_axis_name="s"),
    out_type=jax.ShapeDtypeStruct((NUM_HEADS, SEQ, SEQ), jnp.float32),
    scratch_types=(
        [pltpu.VMEM((TBL_WORDS,), jnp.float32)]
        + [pltpu.VMEM((ROW_PAD,), jnp.float32)] * NB
        + [pltpu.SemaphoreType.DMA] * NB
    ),
    compiler_params=pltpu.CompilerParams(
        needs_layout_passes=False, use_tc_tiling_on_sc=False),
)
def _bias_kernel(tbl_hbm, out_hbm, tbl_v, *bufs_sems):
    bufs, sems = bufs_sems[:NB], bufs_sems[NB:]
    wid = lax.axis_index("s") * 2 + lax.axis_index("c")
    pltpu.sync_copy(tbl_hbm, tbl_v)

    start = (wid * (TOTAL_ROWS - ROWS_PER)) // (NW - 1)
    lane = lax.iota(jnp.int32, 16)
    lane0 = lane == 0
    edge = jnp.where(lane0, 31, 0).astype(jnp.int32)

    def build_row(buf, r):
        h = r // SEQ
        i = r - h * SEQ
        hb = h * 3972

        @pl.when(i == 0)
        def _():
            vspec = plsc.load_gather(
                tbl_v, [jnp.where(lane0, hb + T_CORNER, hb + T_ROW0)])
            buf[pl.ds(0, 16)] = vspec
            vfill = plsc.load_gather(
                tbl_v, [jnp.full((16,), hb + T_ROW0, jnp.int32)])
            for j in range(1, 65):
                buf[pl.ds(16 * j, 16)] = vfill

        @pl.when(i != 0)
        def _():
            p = i - 1
            r0 = p // 32
            c0 = p - r0 * 32
            s = hb + (r0 + 31) * 63 + (c0 + 31)
            base0 = jnp.full((16,), s + 1, jnp.int32) - lane
            o0 = jnp.where(lane0, hb + T_COL0, base0)
            buf[pl.ds(0, 16)] = plsc.load_gather(tbl_v, [o0])
            for j in range(1, 64):
                if j % 2 == 1:
                    off = base0 - (63 * ((j - 1) // 2) + 16)
                else:
                    off = base0 - 63 * (j // 2) + edge
                buf[pl.ds(16 * j, 16)] = plsc.load_gather(tbl_v, [off])
            o64 = jnp.full((16,), s - 1984, jnp.int32)
            buf[pl.ds(1024, 16)] = plsc.load_gather(tbl_v, [o64])

    def do_iter(it, carry):
        for b in range(NB):
            buf, sem = bufs[b], sems[b]

            @pl.when(it > 0)
            def _():
                pltpu.make_async_copy(
                    buf.at[pl.ds(0, SEQ)], out_hbm.at[0, 0], sem).wait()

            r = start + NB * it + b
            build_row(buf, r)
            pltpu.async_copy(
                buf.at[pl.ds(0, SEQ)], out_hbm.at[r // SEQ, r % SEQ], sem)
        return carry

    lax.fori_loop(0, ROWS_PER // NB, do_iter, 0)
    for b in range(NB):
        pltpu.make_async_copy(
            bufs[b].at[pl.ds(0, SEQ)], out_hbm.at[0, 0], sems[b]).wait()


def kernel(relative_position_bias_table, window_size):
    del window_size
    tbl_flat = relative_position_bias_table.T.reshape(-1)
    out = _bias_kernel(tbl_flat)
    return out[None]


# speedup vs baseline: 14.3572x; 2.1503x over previous
import functools

import jax
import jax.numpy as jnp
from jax import lax
from jax.experimental import pallas as pl
from jax.experimental.pallas import tpu as pltpu
from jax.experimental.pallas import tpu_sc as plsc

NUM_HEADS = 16
SEQ = 1025
TBL_WORDS = 3972 * NUM_HEADS
NW = 32
TOTAL_ROWS = NUM_HEADS * SEQ
NB = 4
ROWS_PER = 520
ROW_PAD = 1040

T_ROW0 = 3969
T_COL0 = 3970
T_CORNER = 3971


@functools.partial(
    pl.kernel,
    mesh=plsc.VectorSubcoreMesh(core_axis_name="c", subcore_axis_name="s"),
    out_type=jax.ShapeDtypeStruct((SEQ, NUM_HEADS, SEQ), jnp.float32),
    scratch_types=(
        [pltpu.VMEM((TBL_WORDS,), jnp.float32)]
        + [pltpu.VMEM((ROW_PAD,), jnp.float32)] * NB
        + [pltpu.SemaphoreType.DMA] * NB
    ),
    compiler_params=pltpu.CompilerParams(
        needs_layout_passes=False, use_tc_tiling_on_sc=False,
        disable_bounds_checks=True),
)
def _bias_kernel(tbl_hbm, out_hbm, tbl_v, *bufs_sems):
    bufs, sems = bufs_sems[:NB], bufs_sems[NB:]
    wid = lax.axis_index("s") * 2 + lax.axis_index("c")
    pltpu.sync_copy(tbl_hbm, tbl_v)

    start = (wid * (TOTAL_ROWS - ROWS_PER)) // (NW - 1)
    lane = lax.iota(jnp.int32, 16)
    lane0 = lane == 0
    edge = jnp.where(lane0, 31, 0).astype(jnp.int32)

    def build_row(buf, r):
        h = r // SEQ
        i = r - h * SEQ
        hb = h * 3972

        @pl.when(i == 0)
        def _():
            vspec = plsc.load_gather(
                tbl_v, [jnp.where(lane0, hb + T_CORNER, hb + T_ROW0)])
            buf[pl.ds(0, 16)] = vspec
            vfill = plsc.load_gather(
                tbl_v, [jnp.full((16,), hb + T_ROW0, jnp.int32)])
            for j in range(1, 65):
                buf[pl.ds(16 * j, 16)] = vfill

        @pl.when(i != 0)
        def _():
            p = i - 1
            r0 = p // 32
            c0 = p - r0 * 32
            s = hb + (r0 + 31) * 63 + (c0 + 31)
            base0 = jnp.full((16,), s + 1, jnp.int32) - lane
            o0 = jnp.where(lane0, hb + T_COL0, base0)
            buf[pl.ds(0, 16)] = plsc.load_gather(tbl_v, [o0])
            for j in range(1, 64):
                if j % 2 == 1:
                    off = base0 - (63 * ((j - 1) // 2) + 16)
                else:
                    off = base0 - 63 * (j // 2) + edge
                buf[pl.ds(16 * j, 16)] = plsc.load_gather(tbl_v, [off])
            o64 = jnp.full((16,), s - 1984, jnp.int32)
            buf[pl.ds(1024, 16)] = plsc.load_gather(tbl_v, [o64])

    def do_iter(it, carry):
        for b in range(NB):
            buf, sem = bufs[b], sems[b]

            @pl.when(it > 0)
            def _():
                pltpu.make_async_copy(
                    buf.at[pl.ds(0, SEQ)], out_hbm.at[0, 0], sem).wait()

            r = start + NB * it + b
            build_row(buf, r)
            pltpu.async_copy(
                buf.at[pl.ds(0, SEQ)], out_hbm.at[r % SEQ, r // SEQ], sem)
        return carry

    lax.fori_loop(0, ROWS_PER // NB, do_iter, 0)
    for b in range(NB):
        pltpu.make_async_copy(
            bufs[b].at[pl.ds(0, SEQ)], out_hbm.at[0, 0], sems[b]).wait()


def kernel(relative_position_bias_table, window_size):
    del window_size
    tbl_flat = relative_position_bias_table.T.reshape(-1)
    out = _bias_kernel(tbl_flat)
    return out.transpose(1, 0, 2)[None]
